# Initial kernel scaffold; baseline (speedup 1.0000x reference)
import jax
import jax.numpy as jnp
from jax.experimental import pallas as pl


def kernel(input, table, inv_freq):
    raise NotImplementedError("write your pallas kernel here")


# speedup vs baseline: 12.6706x; 12.6706x over previous
import functools

import jax
import jax.numpy as jnp
from jax import lax
from jax.experimental import pallas as pl
from jax.experimental.pallas import tpu as pltpu
from jax.experimental.pallas import tpu_sc as plsc


def _prescale(table, inv2d):
    V, D = table.shape

    def body(t_ref, w_ref, o_ref):
        rows = lax.broadcasted_iota(jnp.int32, (V, D), 0)
        o_ref[...] = jnp.where(rows == 0, jnp.float32(0.0), t_ref[...] * w_ref[...])

    return pl.pallas_call(
        body,
        out_shape=jax.ShapeDtypeStruct((V, D), table.dtype),
    )(table, inv2d)


def _sc_gather(scaled, idx):
    (N,) = idx.shape
    V, D = scaled.shape
    info = plsc.get_sparse_core_info()
    NC, NS = info.num_cores, info.num_subcores
    NW = NC * NS
    rpw = N // NW
    CHUNK = 128
    n_chunks = rpw // CHUNK
    mesh = plsc.VectorSubcoreMesh(core_axis_name="c", subcore_axis_name="s")

    @functools.partial(
        pl.kernel,
        mesh=mesh,
        out_type=jax.ShapeDtypeStruct((N, D), jnp.float32),
        scratch_types=[
            pltpu.VMEM((rpw,), jnp.int32),
            pltpu.VMEM((CHUNK, D), jnp.float32),
            pltpu.SemaphoreType.DMA,
        ],
        compiler_params=pltpu.CompilerParams(use_tc_tiling_on_sc=False),
    )
    def k(tab_hbm, idx_hbm, out_hbm, idx_v, buf, sem):
        wid = lax.axis_index("s") * NC + lax.axis_index("c")
        base = wid * rpw
        pltpu.sync_copy(idx_hbm.at[pl.ds(base, rpw)], idx_v)

        def step(i, carry):
            off = pl.multiple_of(i * CHUNK, CHUNK)
            pltpu.async_copy(tab_hbm.at[idx_v.at[pl.ds(off, CHUNK)]], buf, sem).wait()
            pltpu.sync_copy(buf, out_hbm.at[pl.ds(base + off, CHUNK)])
            return carry

        lax.fori_loop(0, n_chunks, step, 0)

    return k(scaled, idx)


def kernel(input, table, inv_freq):
    B, T = input.shape
    V, D = table.shape
    scaled = _prescale(table, inv_freq.reshape(V, 1))
    out = _sc_gather(scaled, input.reshape(-1))
    return out.reshape(B, T, D)


# speedup vs baseline: 13.3161x; 1.0509x over previous
import functools

import jax
import jax.numpy as jnp
from jax import lax
from jax.experimental import pallas as pl
from jax.experimental.pallas import tpu as pltpu
from jax.experimental.pallas import tpu_sc as plsc


def _prescale(table, inv2d):
    V, D = table.shape

    def body(t_ref, w_ref, o_ref):
        rows = lax.broadcasted_iota(jnp.int32, (V, D), 0)
        o_ref[...] = jnp.where(rows == 0, jnp.float32(0.0), t_ref[...] * w_ref[...])

    return pl.pallas_call(
        body,
        out_shape=jax.ShapeDtypeStruct((V, D), table.dtype),
    )(table, inv2d)


def _sc_gather(scaled, idx):
    (N,) = idx.shape
    V, D = scaled.shape
    info = plsc.get_sparse_core_info()
    NC, NS = info.num_cores, info.num_subcores
    NW = NC * NS
    rpw = N // NW
    CHUNK = 128
    NBUF = 4
    n_chunks = rpw // CHUNK
    n_outer = n_chunks // NBUF
    mesh = plsc.VectorSubcoreMesh(core_axis_name="c", subcore_axis_name="s")

    @functools.partial(
        pl.kernel,
        mesh=mesh,
        out_type=jax.ShapeDtypeStruct((N, D), jnp.float32),
        scratch_types=[
            pltpu.VMEM((rpw,), jnp.int32),
            pltpu.VMEM((NBUF, CHUNK, D), jnp.float32),
        ] + [pltpu.SemaphoreType.DMA] * NBUF,
        compiler_params=pltpu.CompilerParams(use_tc_tiling_on_sc=False),
    )
    def k(tab_hbm, idx_hbm, out_hbm, idx_v, buf, *sems):
        wid = lax.axis_index("s") * NC + lax.axis_index("c")
        base = wid * rpw
        pltpu.sync_copy(idx_hbm.at[pl.ds(base, rpw)], idx_v)

        def gather(c, b):
            off = pl.multiple_of(c * CHUNK, CHUNK)
            return pltpu.make_async_copy(
                tab_hbm.at[idx_v.at[pl.ds(off, CHUNK)]], buf.at[b], sems[b])

        for b in range(NBUF):
            gather(b, b).start()

        def outer(i, carry):
            for b in range(NBUF):
                c = i * NBUF + b
                gather(c, b).wait()
                off = pl.multiple_of(c * CHUNK, CHUNK)
                pltpu.sync_copy(buf.at[b], out_hbm.at[pl.ds(base + off, CHUNK)])

                @pl.when(c + NBUF < n_chunks)
                def _():
                    gather(c + NBUF, b).start()

            return carry

        lax.fori_loop(0, n_outer, outer, 0)

    return k(scaled, idx)


def kernel(input, table, inv_freq):
    B, T = input.shape
    V, D = table.shape
    scaled = _prescale(table, inv_freq.reshape(V, 1))
    out = _sc_gather(scaled, input.reshape(-1))
    return out.reshape(B, T, D)
